# initial kernel scaffold (unmeasured)
import jax
import jax.numpy as jnp
from jax import lax
from jax.experimental import pallas as pl
from jax.experimental.pallas import tpu as pltpu

N_DEV = 4
SQ = 1024
SKV = 1024
D_MODEL = 1024
HB = 8
DH = 128
BLK = HB * DH
SCALE = 0.08838834764831843
QC = 512

_F32 = jnp.float32


def _body(x_ref, wq_ref, k_hbm, v_hbm, wo_ref, out_ref,
          wq_buf, wo_buf, k_buf, v_buf, maskb,
          send_sems, recv_sems, kv_sems):
    my = lax.axis_index("i")
    left = lax.rem(my + (N_DEV - 1), N_DEV)
    right = lax.rem(my + 1, N_DEV)

    barrier = pltpu.get_barrier_semaphore()
    for nbr in (left, right):
        pl.semaphore_signal(barrier, inc=1, device_id=(nbr,),
                            device_id_type=pl.DeviceIdType.MESH)
    pl.semaphore_wait(barrier, 2)

    qi = lax.broadcasted_iota(jnp.int32, (SQ, SKV), 0)
    ki = lax.broadcasted_iota(jnp.int32, (SQ, SKV), 1)
    maskb[...] = jnp.where((qi // 64) % 4 == (ki // 64) % 4,
                           0.0, -1e9).astype(_F32)

    out_ref[...] = jnp.zeros((SQ, D_MODEL), _F32)

    send_descs = []
    for h in range(N_DEV):
        j = lax.rem(my - h + N_DEV, N_DEV)
        cur_wq = wq_ref if h == 0 else wq_buf.at[h - 1]
        cur_wo = wo_ref if h == 0 else wo_buf.at[h - 1]

        kcp = pltpu.make_async_copy(
            k_hbm.at[:, pl.ds(j * BLK, BLK)], k_buf, kv_sems.at[0])
        vcp = pltpu.make_async_copy(
            v_hbm.at[:, pl.ds(j * BLK, BLK)], v_buf, kv_sems.at[1])
        kcp.start()
        vcp.start()

        if h < N_DEV - 1:
            r_wq = pltpu.make_async_remote_copy(
                src_ref=cur_wq, dst_ref=wq_buf.at[h],
                send_sem=send_sems.at[0, h], recv_sem=recv_sems.at[0, h],
                device_id=(right,), device_id_type=pl.DeviceIdType.MESH)
            r_wo = pltpu.make_async_remote_copy(
                src_ref=cur_wo, dst_ref=wo_buf.at[h],
                send_sem=send_sems.at[1, h], recv_sem=recv_sems.at[1, h],
                device_id=(right,), device_id_type=pl.DeviceIdType.MESH)
            r_wq.start()
            r_wo.start()
            send_descs.append(r_wq)
            send_descs.append(r_wo)

        kcp.wait()
        vcp.wait()

        wq_c = wq_ref[...] if h == 0 else wq_buf[h - 1]
        q_full = lax.dot_general(x_ref[...], wq_c,
                                 (((1,), (0,)), ((), ())),
                                 preferred_element_type=_F32)

        for hd in range(HB):
            kh = k_buf[:, hd * DH:(hd + 1) * DH]
            vh = v_buf[:, hd * DH:(hd + 1) * DH]
            if h == 0:
                woh = wo_ref[hd * DH:(hd + 1) * DH, :]
            else:
                woh = wo_buf[h - 1, hd * DH:(hd + 1) * DH, :]
            for qc in range(SQ // QC):
                qs = qc * QC
                qh = q_full[qs:qs + QC, hd * DH:(hd + 1) * DH]
                s = lax.dot_general(qh, kh, (((1,), (1,)), ((), ())),
                                    preferred_element_type=_F32)
                s = s * SCALE + maskb[qs:qs + QC, :]
                m = jnp.max(s, axis=1, keepdims=True)
                p = jnp.exp(s - m)
                p = p / jnp.sum(p, axis=1, keepdims=True)
                ctx = lax.dot_general(p, vh, (((1,), (0,)), ((), ())),
                                      preferred_element_type=_F32)
                out_ref[qs:qs + QC, :] += lax.dot_general(
                    ctx, woh, (((1,), (0,)), ((), ())),
                    preferred_element_type=_F32)

        if h < N_DEV - 1:
            r_wq.wait_recv()
            r_wo.wait_recv()

    for d in send_descs:
        d.wait_send()


def kernel(x, Wq, K_ext, V_ext, Wo):
    x2 = x.reshape(SQ, D_MODEL)
    k2 = K_ext.reshape(SKV, N_DEV * BLK)
    v2 = V_ext.reshape(SKV, N_DEV * BLK)
    out = pl.pallas_call(
        _body,
        out_shape=jax.ShapeDtypeStruct((SQ, D_MODEL), _F32),
        in_specs=[
            pl.BlockSpec(memory_space=pltpu.VMEM),
            pl.BlockSpec(memory_space=pltpu.VMEM),
            pl.BlockSpec(memory_space=pltpu.ANY),
            pl.BlockSpec(memory_space=pltpu.ANY),
            pl.BlockSpec(memory_space=pltpu.VMEM),
        ],
        out_specs=pl.BlockSpec(memory_space=pltpu.VMEM),
        scratch_shapes=[
            pltpu.VMEM((N_DEV - 1, D_MODEL, BLK), _F32),
            pltpu.VMEM((N_DEV - 1, BLK, D_MODEL), _F32),
            pltpu.VMEM((SKV, BLK), _F32),
            pltpu.VMEM((SKV, BLK), _F32),
            pltpu.VMEM((SQ, SKV), _F32),
            pltpu.SemaphoreType.DMA((2, N_DEV - 1)),
            pltpu.SemaphoreType.DMA((2, N_DEV - 1)),
            pltpu.SemaphoreType.DMA((2,)),
        ],
        compiler_params=pltpu.CompilerParams(collective_id=0),
    )(x2, Wq, k2, v2, Wo)
    return out.reshape(1, SQ, D_MODEL)


# baseline (device time: 224209 ns/iter reference)
import jax
import jax.numpy as jnp
from jax import lax
from jax.experimental import pallas as pl
from jax.experimental.pallas import tpu as pltpu

N_DEV = 4
SQ = 1024
SKV = 1024
D_MODEL = 1024
HB = 8
DH = 128
BLK = HB * DH
SCALE = 0.08838834764831843
QC = 512

_F32 = jnp.float32
_BF16 = jnp.bfloat16


def _body(x_ref, wq_ref, k_hbm, v_hbm, wo_ref, out_ref,
          wq_buf, wo_buf, k_buf, v_buf, maskb,
          send_sems, recv_sems, kv_sems):
    my = lax.axis_index("i")
    left = lax.rem(my + (N_DEV - 1), N_DEV)
    right = lax.rem(my + 1, N_DEV)

    barrier = pltpu.get_barrier_semaphore()
    for nbr in (left, right):
        pl.semaphore_signal(barrier, inc=1, device_id=(nbr,),
                            device_id_type=pl.DeviceIdType.MESH)
    pl.semaphore_wait(barrier, 2)

    qi = lax.broadcasted_iota(jnp.int32, (SQ, SKV), 0)
    ki = lax.broadcasted_iota(jnp.int32, (SQ, SKV), 1)
    maskb[...] = jnp.where((qi // 64) % 4 == (ki // 64) % 4,
                           0.0, -1e9).astype(_F32)

    out_ref[...] = jnp.zeros((SQ, D_MODEL), _F32)

    send_descs = []
    for h in range(N_DEV):
        j = lax.rem(my - h + N_DEV, N_DEV)
        cur_wq = wq_ref if h == 0 else wq_buf.at[h - 1]
        cur_wo = wo_ref if h == 0 else wo_buf.at[h - 1]

        kcp = pltpu.make_async_copy(
            k_hbm.at[:, pl.ds(j * BLK, BLK)], k_buf, kv_sems.at[0])
        vcp = pltpu.make_async_copy(
            v_hbm.at[:, pl.ds(j * BLK, BLK)], v_buf, kv_sems.at[1])
        kcp.start()
        vcp.start()

        if h < N_DEV - 1:
            r_wq = pltpu.make_async_remote_copy(
                src_ref=cur_wq, dst_ref=wq_buf.at[h],
                send_sem=send_sems.at[0, h], recv_sem=recv_sems.at[0, h],
                device_id=(right,), device_id_type=pl.DeviceIdType.MESH)
            r_wo = pltpu.make_async_remote_copy(
                src_ref=cur_wo, dst_ref=wo_buf.at[h],
                send_sem=send_sems.at[1, h], recv_sem=recv_sems.at[1, h],
                device_id=(right,), device_id_type=pl.DeviceIdType.MESH)
            r_wq.start()
            r_wo.start()
            send_descs.append(r_wq)
            send_descs.append(r_wo)

        kcp.wait()
        vcp.wait()

        wq_c = wq_ref[...] if h == 0 else wq_buf[h - 1]
        q_full = lax.dot_general(x_ref[...], wq_c,
                                 (((1,), (0,)), ((), ())),
                                 preferred_element_type=_F32)

        for hd in range(HB):
            kh = k_buf[:, hd * DH:(hd + 1) * DH]
            vh = v_buf[:, hd * DH:(hd + 1) * DH]
            if h == 0:
                woh = wo_ref[hd * DH:(hd + 1) * DH, :]
            else:
                woh = wo_buf[h - 1, hd * DH:(hd + 1) * DH, :]
            for qc in range(SQ // QC):
                qs = qc * QC
                qh = q_full[qs:qs + QC, hd * DH:(hd + 1) * DH].astype(_BF16)
                s = lax.dot_general(qh, kh, (((1,), (1,)), ((), ())),
                                    preferred_element_type=_F32)
                s = s * SCALE + maskb[qs:qs + QC, :]
                m = jnp.max(s, axis=1, keepdims=True)
                p = jnp.exp(s - m)
                p = (p / jnp.sum(p, axis=1, keepdims=True)).astype(_BF16)
                ctx = lax.dot_general(p, vh, (((1,), (0,)), ((), ())),
                                      preferred_element_type=_F32)
                out_ref[qs:qs + QC, :] += lax.dot_general(
                    ctx.astype(_BF16), woh, (((1,), (0,)), ((), ())),
                    preferred_element_type=_F32)

        if h < N_DEV - 1:
            r_wq.wait_recv()
            r_wo.wait_recv()

    for d in send_descs:
        d.wait_send()


def kernel(x, Wq, K_ext, V_ext, Wo):
    x2 = x.reshape(SQ, D_MODEL).astype(_BF16)
    wq = Wq.astype(_BF16)
    wo = Wo.astype(_BF16)
    k2 = K_ext.reshape(SKV, N_DEV * BLK).astype(_BF16)
    v2 = V_ext.reshape(SKV, N_DEV * BLK).astype(_BF16)
    out = pl.pallas_call(
        _body,
        out_shape=jax.ShapeDtypeStruct((SQ, D_MODEL), _F32),
        in_specs=[
            pl.BlockSpec(memory_space=pltpu.VMEM),
            pl.BlockSpec(memory_space=pltpu.VMEM),
            pl.BlockSpec(memory_space=pl.ANY),
            pl.BlockSpec(memory_space=pl.ANY),
            pl.BlockSpec(memory_space=pltpu.VMEM),
        ],
        out_specs=pl.BlockSpec(memory_space=pltpu.VMEM),
        scratch_shapes=[
            pltpu.VMEM((N_DEV - 1, D_MODEL, BLK), _BF16),
            pltpu.VMEM((N_DEV - 1, BLK, D_MODEL), _BF16),
            pltpu.VMEM((SKV, BLK), _BF16),
            pltpu.VMEM((SKV, BLK), _BF16),
            pltpu.VMEM((SQ, SKV), _F32),
            pltpu.SemaphoreType.DMA((2, N_DEV - 1)),
            pltpu.SemaphoreType.DMA((2, N_DEV - 1)),
            pltpu.SemaphoreType.DMA((2,)),
        ],
        compiler_params=pltpu.CompilerParams(
            collective_id=0, vmem_limit_bytes=48 * 1024 * 1024),
    )(x2, wq, k2, v2, wo)
    return out.reshape(1, SQ, D_MODEL)


# device time: 128932 ns/iter; 1.7390x vs baseline; 1.7390x over previous
import jax
import jax.numpy as jnp
from jax import lax
from jax.experimental import pallas as pl
from jax.experimental.pallas import tpu as pltpu

N_DEV = 4
SQ = 1024
SKV = 1024
D_MODEL = 1024
HB = 8
DH = 128
BLK = HB * DH
G = 4
GS = SQ // G
SCALE = 0.08838834764831843

_F32 = jnp.float32
_BF16 = jnp.bfloat16

_MESH = pl.DeviceIdType.MESH


def _body(x_ref, wq_ref, k_hbm, v_hbm, wo_ref, out_ref,
          wq_buf, wo_buf, k_buf, v_buf, ctx_hop, ctx_keep,
          send_sems, recv_sems, kv_sems):
    my = lax.axis_index("i")
    left = lax.rem(my + (N_DEV - 1), N_DEV)
    right = lax.rem(my + 1, N_DEV)

    barrier = pltpu.get_barrier_semaphore()
    for nbr in (left, right):
        pl.semaphore_signal(barrier, inc=1, device_id=(nbr,),
                            device_id_type=_MESH)
    pl.semaphore_wait(barrier, 2)

    def wq_send(h, src):
        d = pltpu.make_async_remote_copy(
            src_ref=src, dst_ref=wq_buf.at[h],
            send_sem=send_sems.at[0, h], recv_sem=recv_sems.at[0, h],
            device_id=(right,), device_id_type=_MESH)
        d.start()
        return d

    def wo_send(h, src):
        d = pltpu.make_async_remote_copy(
            src_ref=src, dst_ref=wo_buf.at[h],
            send_sem=send_sems.at[1, h], recv_sem=recv_sems.at[1, h],
            device_id=(left,), device_id_type=_MESH)
        d.start()
        return d

    def kv_start(slot, blk):
        kcp = pltpu.make_async_copy(
            k_hbm.at[:, pl.ds(blk * BLK, BLK)], k_buf.at[slot],
            kv_sems.at[0, slot])
        vcp = pltpu.make_async_copy(
            v_hbm.at[:, pl.ds(blk * BLK, BLK)], v_buf.at[slot],
            kv_sems.at[1, slot])
        kcp.start()
        vcp.start()
        return kcp, vcp

    def kv_wait(pair):
        pair[0].wait()
        pair[1].wait()

    def attn(wq_src, slot, ctx_dst):
        q_full = lax.dot_general(x_ref[...], wq_src,
                                 (((1,), (0,)), ((), ())),
                                 preferred_element_type=_F32)
        for hd in range(HB):
            c = hd * DH
            q3 = q_full[:, c:c + DH].astype(_BF16).reshape(G, GS, DH)
            k3 = k_buf[slot, :, c:c + DH].reshape(G, GS, DH)
            v3 = v_buf[slot, :, c:c + DH].reshape(G, GS, DH)
            s = lax.dot_general(q3, k3, (((2,), (2,)), ((0,), (0,))),
                                preferred_element_type=_F32) * SCALE
            m = jnp.max(s, axis=2, keepdims=True)
            p = jnp.exp(s - m)
            p = (p / jnp.sum(p, axis=2, keepdims=True)).astype(_BF16)
            ctx = lax.dot_general(p, v3, (((2,), (1,)), ((0,), (0,))),
                                  preferred_element_type=_F32)
            ctx_dst[:, c:c + DH] = ctx.astype(_BF16).reshape(SQ, DH)

    def outproj(ctx_src, wo_src, first=False):
        y = lax.dot_general(ctx_src[...], wo_src,
                            (((1,), (0,)), ((), ())),
                            preferred_element_type=_F32)
        if first:
            out_ref[...] = y
        else:
            out_ref[...] += y

    snd = [wq_send(0, wq_ref), wo_send(0, wo_ref)]
    kv0 = kv_start(0, my)
    kv1 = kv_start(1, lax.rem(my + 3, N_DEV))

    kv_wait(kv0)
    attn(wq_ref[...], 0, ctx_hop)
    outproj(ctx_hop, wo_ref[...], first=True)

    snd[0].wait_recv()
    snd[1].wait_recv()
    snd.append(wq_send(1, wq_buf.at[0]))
    snd.append(wo_send(1, wo_buf.at[0]))
    kv2 = kv_start(0, lax.rem(my + 2, N_DEV))
    kv_wait(kv1)
    attn(wq_buf[0], 1, ctx_keep)

    snd[2].wait_recv()
    snd[3].wait_recv()
    snd.append(wq_send(2, wq_buf.at[1]))
    snd.append(wo_send(2, wo_buf.at[1]))
    kv3 = kv_start(1, lax.rem(my + 1, N_DEV))
    kv_wait(kv2)
    attn(wq_buf[1], 0, ctx_hop)
    outproj(ctx_hop, wo_buf[1])

    snd[4].wait_recv()
    snd[5].wait_recv()
    kv_wait(kv3)
    attn(wq_buf[2], 1, ctx_hop)
    outproj(ctx_hop, wo_buf[0])
    outproj(ctx_keep, wo_buf[2])

    for d in snd:
        d.wait_send()


def _perm(a):
    n = a.shape[0]
    return a.reshape(G, G, 64, -1).transpose(1, 0, 2, 3).reshape(n, -1)


def kernel(x, Wq, K_ext, V_ext, Wo):
    x2 = _perm(x.reshape(SQ, D_MODEL)).astype(_BF16)
    wq = Wq.astype(_BF16)
    wo = Wo.astype(_BF16)
    k2 = _perm(K_ext.reshape(SKV, N_DEV * BLK)).astype(_BF16)
    v2 = _perm(V_ext.reshape(SKV, N_DEV * BLK)).astype(_BF16)
    out = pl.pallas_call(
        _body,
        out_shape=jax.ShapeDtypeStruct((SQ, D_MODEL), _F32),
        in_specs=[
            pl.BlockSpec(memory_space=pltpu.VMEM),
            pl.BlockSpec(memory_space=pltpu.VMEM),
            pl.BlockSpec(memory_space=pl.ANY),
            pl.BlockSpec(memory_space=pl.ANY),
            pl.BlockSpec(memory_space=pltpu.VMEM),
        ],
        out_specs=pl.BlockSpec(memory_space=pltpu.VMEM),
        scratch_shapes=[
            pltpu.VMEM((N_DEV - 1, D_MODEL, BLK), _BF16),
            pltpu.VMEM((N_DEV - 1, BLK, D_MODEL), _BF16),
            pltpu.VMEM((2, SKV, BLK), _BF16),
            pltpu.VMEM((2, SKV, BLK), _BF16),
            pltpu.VMEM((SQ, BLK), _BF16),
            pltpu.VMEM((SQ, BLK), _BF16),
            pltpu.SemaphoreType.DMA((2, N_DEV - 1)),
            pltpu.SemaphoreType.DMA((2, N_DEV - 1)),
            pltpu.SemaphoreType.DMA((2, 2)),
        ],
        compiler_params=pltpu.CompilerParams(
            collective_id=0, vmem_limit_bytes=48 * 1024 * 1024),
    )(x2, wq, k2, v2, wo)
    return _perm(out).reshape(1, SQ, D_MODEL)
